# baseline (speedup 1.0000x reference)
import functools

import jax
import jax.numpy as jnp
from jax.experimental import pallas as pl

N = 2048
_DOT = functools.partial(jnp.dot, preferred_element_type=jnp.float32)


def _gcn3_body(x_ref, a_ref, w1_ref, w2_ref, w3_ref, out_ref):
    a = a_ref[:]
    o = jnp.maximum(_DOT(a, _DOT(x_ref[:], w1_ref[:])), 0.0)
    o = jnp.maximum(_DOT(a, _DOT(o, w2_ref[:])), 0.0)
    out_ref[:] = jnp.maximum(_DOT(a, _DOT(o, w3_ref[:])), 0.0)


def _gcn3(x, a, w1, w2, w3):
    return pl.pallas_call(
        _gcn3_body,
        out_shape=jax.ShapeDtypeStruct((N, w3.shape[1]), jnp.float32),
    )(x, a, w1, w2, w3)


def _combine_body(h1_ref, h2_ref, h3_ref, alpha_ref, out_ref):
    alpha = alpha_ref[0, 0]
    out_ref[:] = alpha * 0.5 * (h1_ref[:] + h2_ref[:]) + (1.0 - alpha) * h3_ref[:]


def _combine(h1, h2, h3, alpha):
    return pl.pallas_call(
        _combine_body,
        out_shape=jax.ShapeDtypeStruct(h1.shape, jnp.float32),
    )(h1, h2, h3, alpha.reshape(1, 1))


_S_BLK = 256


def _s_body(hi_ref, hjt_ref, xi_ref, xjt_ref, out_ref):
    s_enc = jax.nn.sigmoid(_DOT(hi_ref[:], hjt_ref[:]))
    s_dec = jax.nn.sigmoid(_DOT(xi_ref[:], xjt_ref[:]))
    out_ref[:] = 0.5 * (s_enc + s_dec)


def _s_avg(h_enc, h_enc_t, x_dec, x_dec_t):
    nb = N // _S_BLK
    kh = h_enc.shape[1]
    kx = x_dec.shape[1]
    return pl.pallas_call(
        _s_body,
        grid=(nb, nb),
        in_specs=[
            pl.BlockSpec((_S_BLK, kh), lambda i, j: (i, 0)),
            pl.BlockSpec((kh, _S_BLK), lambda i, j: (0, j)),
            pl.BlockSpec((_S_BLK, kx), lambda i, j: (i, 0)),
            pl.BlockSpec((kx, _S_BLK), lambda i, j: (0, j)),
        ],
        out_specs=pl.BlockSpec((_S_BLK, _S_BLK), lambda i, j: (i, j)),
        out_shape=jax.ShapeDtypeStruct((N, N), jnp.float32),
    )(h_enc, h_enc_t, x_dec, x_dec_t)


def kernel(x, x_mask, A1, A2, G, Wg_e1, Wg_e2, Wg_e3, Wg_d1, Wg_d2, Wg_d3,
           Wh_e1, Wh_e2, Wh_e3, Wh_d1, Wh_d2, Wh_d3, alpha):
    h1 = _gcn3(x, A1, Wg_e1, Wg_e2, Wg_e3)
    h2 = _gcn3(x_mask, A2, Wg_e1, Wg_e2, Wg_e3)
    h3 = _gcn3(x, G, Wh_e1, Wh_e2, Wh_e3)
    h = _combine(h1, h2, h3, alpha)
    x1 = _gcn3(h, A1, Wg_d1, Wg_d2, Wg_d3)
    x2 = _gcn3(h, A2, Wg_d1, Wg_d2, Wg_d3)
    x3 = _gcn3(h, G, Wh_d1, Wh_d2, Wh_d3)
    s1 = _s_avg(h1, h1.T, x1, x1.T)
    s2 = _s_avg(h2, h2.T, x2, x2.T)
    s3 = _s_avg(h3, h3.T, x3, x3.T)
    return (h, s1, s2, s3, x1, x2, x3)


# speedup vs baseline: 1.2726x; 1.2726x over previous
import functools

import jax
import jax.numpy as jnp
from jax.experimental import pallas as pl

N = 2048
_DOT = functools.partial(jnp.dot, preferred_element_type=jnp.float32)
_BF = jnp.bfloat16


def _gcn3_body(x_ref, a_ref, w1_ref, w2_ref, w3_ref, out_ref):
    a = a_ref[:]
    u = _DOT(x_ref[:], w1_ref[:].astype(_BF)).astype(_BF)
    o = jnp.maximum(_DOT(a, u), 0.0).astype(_BF)
    o = jnp.maximum(_DOT(a, _DOT(o, w2_ref[:].astype(_BF)).astype(_BF)), 0.0).astype(_BF)
    out_ref[:] = jnp.maximum(_DOT(a, _DOT(o, w3_ref[:].astype(_BF)).astype(_BF)), 0.0)


def _gcn3(x_bf, a_bf, w1, w2, w3):
    return pl.pallas_call(
        _gcn3_body,
        out_shape=jax.ShapeDtypeStruct((N, w3.shape[1]), jnp.float32),
    )(x_bf, a_bf, w1, w2, w3)


def _combine_body(h1_ref, h2_ref, h3_ref, alpha_ref, out_ref):
    alpha = alpha_ref[0, 0]
    out_ref[:] = alpha * 0.5 * (h1_ref[:] + h2_ref[:]) + (1.0 - alpha) * h3_ref[:]


def _combine(h1, h2, h3, alpha):
    return pl.pallas_call(
        _combine_body,
        out_shape=jax.ShapeDtypeStruct(h1.shape, jnp.float32),
    )(h1, h2, h3, alpha.reshape(1, 1))


_S_BLK = 512


def _s_body(hi_ref, hjt_ref, xi_ref, xjt_ref, out_ref):
    s_enc = jax.nn.sigmoid(_DOT(hi_ref[:].astype(_BF), hjt_ref[:].astype(_BF)))
    s_dec = jax.nn.sigmoid(_DOT(xi_ref[:].astype(_BF), xjt_ref[:].astype(_BF)))
    out_ref[:] = 0.5 * (s_enc + s_dec)


def _s_avg(h_enc, h_enc_t, x_dec, x_dec_t):
    nb = N // _S_BLK
    kh = h_enc.shape[1]
    kx = x_dec.shape[1]
    return pl.pallas_call(
        _s_body,
        grid=(nb, nb),
        in_specs=[
            pl.BlockSpec((_S_BLK, kh), lambda i, j: (i, 0)),
            pl.BlockSpec((kh, _S_BLK), lambda i, j: (0, j)),
            pl.BlockSpec((_S_BLK, kx), lambda i, j: (i, 0)),
            pl.BlockSpec((kx, _S_BLK), lambda i, j: (0, j)),
        ],
        out_specs=pl.BlockSpec((_S_BLK, _S_BLK), lambda i, j: (i, j)),
        out_shape=jax.ShapeDtypeStruct((N, N), jnp.float32),
    )(h_enc, h_enc_t, x_dec, x_dec_t)


def kernel(x, x_mask, A1, A2, G, Wg_e1, Wg_e2, Wg_e3, Wg_d1, Wg_d2, Wg_d3,
           Wh_e1, Wh_e2, Wh_e3, Wh_d1, Wh_d2, Wh_d3, alpha):
    a1_bf = A1.astype(_BF)
    a2_bf = A2.astype(_BF)
    g_bf = G.astype(_BF)
    x_bf = x.astype(_BF)
    xm_bf = x_mask.astype(_BF)
    h1 = _gcn3(x_bf, a1_bf, Wg_e1, Wg_e2, Wg_e3)
    h2 = _gcn3(xm_bf, a2_bf, Wg_e1, Wg_e2, Wg_e3)
    h3 = _gcn3(x_bf, g_bf, Wh_e1, Wh_e2, Wh_e3)
    h = _combine(h1, h2, h3, alpha)
    h_bf = h.astype(_BF)
    x1 = _gcn3(h_bf, a1_bf, Wg_d1, Wg_d2, Wg_d3)
    x2 = _gcn3(h_bf, a2_bf, Wg_d1, Wg_d2, Wg_d3)
    x3 = _gcn3(h_bf, g_bf, Wh_d1, Wh_d2, Wh_d3)
    s1 = _s_avg(h1, h1.T, x1, x1.T)
    s2 = _s_avg(h2, h2.T, x2, x2.T)
    s3 = _s_avg(h3, h3.T, x3, x3.T)
    return (h, s1, s2, s3, x1, x2, x3)


# speedup vs baseline: 1.5553x; 1.2221x over previous
import functools

import jax
import jax.numpy as jnp
from jax.experimental import pallas as pl

N = 2048
_DOT = functools.partial(jnp.dot, preferred_element_type=jnp.float32)
_BF = jnp.bfloat16


def _dot_nt(a, b):
    return jax.lax.dot_general(a, b, (((1,), (1,)), ((), ())),
                               preferred_element_type=jnp.float32)


def _gcn3(x, a, w1, w2, w3):
    u = _DOT(x, w1.astype(_BF)).astype(_BF)
    o = jnp.maximum(_DOT(a, u), 0.0).astype(_BF)
    o = jnp.maximum(_DOT(a, _DOT(o, w2.astype(_BF)).astype(_BF)), 0.0).astype(_BF)
    return jnp.maximum(_DOT(a, _DOT(o, w3.astype(_BF)).astype(_BF)), 0.0)


def _enc_body(x_ref, xm_ref, a1_ref, a2_ref, g_ref,
              wge1_ref, wge2_ref, wge3_ref,
              whe1_ref, whe2_ref, whe3_ref,
              alpha_ref,
              h_ref, hbf_ref, h1_ref, h2_ref, h3_ref):
    h1 = _gcn3(x_ref[:], a1_ref[:], wge1_ref[:], wge2_ref[:], wge3_ref[:])
    h2 = _gcn3(xm_ref[:], a2_ref[:], wge1_ref[:], wge2_ref[:], wge3_ref[:])
    h3 = _gcn3(x_ref[:], g_ref[:], whe1_ref[:], whe2_ref[:], whe3_ref[:])
    alpha = alpha_ref[0, 0]
    h = alpha * 0.5 * (h1 + h2) + (1.0 - alpha) * h3
    h_ref[:] = h
    hbf_ref[:] = h.astype(_BF)
    h1_ref[:] = h1.astype(_BF)
    h2_ref[:] = h2.astype(_BF)
    h3_ref[:] = h3.astype(_BF)


def _enc(x_bf, xm_bf, a1_bf, a2_bf, g_bf, ws, alpha):
    f32 = jnp.float32
    out_shapes = (
        jax.ShapeDtypeStruct((N, 32), f32),
        jax.ShapeDtypeStruct((N, 32), _BF),
        jax.ShapeDtypeStruct((N, 32), _BF),
        jax.ShapeDtypeStruct((N, 32), _BF),
        jax.ShapeDtypeStruct((N, 32), _BF),
    )
    return pl.pallas_call(
        _enc_body,
        out_shape=out_shapes,
    )(x_bf, xm_bf, a1_bf, a2_bf, g_bf, ws[0], ws[1], ws[2], ws[6], ws[7], ws[8],
      alpha.reshape(1, 1))


def _dec_body(hbf_ref, a1_ref, a2_ref, g_ref,
              wgd1_ref, wgd2_ref, wgd3_ref,
              whd1_ref, whd2_ref, whd3_ref,
              x1_ref, x2_ref, x3_ref):
    h_bf = hbf_ref[:]
    x1_ref[:] = _gcn3(h_bf, a1_ref[:], wgd1_ref[:], wgd2_ref[:], wgd3_ref[:])
    x2_ref[:] = _gcn3(h_bf, a2_ref[:], wgd1_ref[:], wgd2_ref[:], wgd3_ref[:])
    x3_ref[:] = _gcn3(h_bf, g_ref[:], whd1_ref[:], whd2_ref[:], whd3_ref[:])


def _dec(h_bf, a1_bf, a2_bf, g_bf, ws):
    f32 = jnp.float32
    out_shapes = (
        jax.ShapeDtypeStruct((N, 256), f32),
        jax.ShapeDtypeStruct((N, 256), f32),
        jax.ShapeDtypeStruct((N, 256), f32),
    )
    return pl.pallas_call(
        _dec_body,
        out_shape=out_shapes,
    )(h_bf, a1_bf, a2_bf, g_bf, ws[3], ws[4], ws[5], ws[9], ws[10], ws[11])


_S_BLK = 512


def _s_body(hi_ref, hj_ref, xi_ref, xj_ref, out_ref):
    s_enc = jax.nn.sigmoid(_dot_nt(hi_ref[:], hj_ref[:]))
    s_dec = jax.nn.sigmoid(_dot_nt(xi_ref[:].astype(_BF), xj_ref[:].astype(_BF)))
    out_ref[:] = 0.5 * (s_enc + s_dec)


def _s_avg(h_enc, x_dec):
    nb = N // _S_BLK
    kh = h_enc.shape[1]
    kx = x_dec.shape[1]
    return pl.pallas_call(
        _s_body,
        grid=(nb, nb),
        in_specs=[
            pl.BlockSpec((_S_BLK, kh), lambda i, j: (i, 0)),
            pl.BlockSpec((_S_BLK, kh), lambda i, j: (j, 0)),
            pl.BlockSpec((_S_BLK, kx), lambda i, j: (i, 0)),
            pl.BlockSpec((_S_BLK, kx), lambda i, j: (j, 0)),
        ],
        out_specs=pl.BlockSpec((_S_BLK, _S_BLK), lambda i, j: (i, j)),
        out_shape=jax.ShapeDtypeStruct((N, N), jnp.float32),
    )(h_enc, h_enc, x_dec, x_dec)


def kernel(x, x_mask, A1, A2, G, Wg_e1, Wg_e2, Wg_e3, Wg_d1, Wg_d2, Wg_d3,
           Wh_e1, Wh_e2, Wh_e3, Wh_d1, Wh_d2, Wh_d3, alpha):
    ws = (Wg_e1, Wg_e2, Wg_e3, Wg_d1, Wg_d2, Wg_d3,
          Wh_e1, Wh_e2, Wh_e3, Wh_d1, Wh_d2, Wh_d3)
    a1_bf = A1.astype(_BF)
    a2_bf = A2.astype(_BF)
    g_bf = G.astype(_BF)
    h, h_bf, h1, h2, h3 = _enc(
        x.astype(_BF), x_mask.astype(_BF), a1_bf, a2_bf, g_bf, ws, alpha)
    x1, x2, x3 = _dec(h_bf, a1_bf, a2_bf, g_bf, ws)
    s1 = _s_avg(h1, x1)
    s2 = _s_avg(h2, x2)
    s3 = _s_avg(h3, x3)
    return (h, s1, s2, s3, x1, x2, x3)


# speedup vs baseline: 1.6530x; 1.0629x over previous
import functools

import jax
import jax.numpy as jnp
from jax.experimental import pallas as pl

N = 2048
_DOT = functools.partial(jnp.dot, preferred_element_type=jnp.float32)
_BF = jnp.bfloat16


def _dot_nt(a, b):
    return jax.lax.dot_general(a, b, (((1,), (1,)), ((), ())),
                               preferred_element_type=jnp.float32)


def _gcn3(x, a, w1, w2, w3):
    u = _DOT(x, w1.astype(_BF)).astype(_BF)
    o = jnp.maximum(_DOT(a, u), 0.0).astype(_BF)
    o = jnp.maximum(_DOT(a, _DOT(o, w2.astype(_BF)).astype(_BF)), 0.0).astype(_BF)
    return jnp.maximum(_DOT(a, _DOT(o, w3.astype(_BF)).astype(_BF)), 0.0)


def _main_body(x_ref, xm_ref, a1_ref, a2_ref, g_ref,
               wge1_ref, wge2_ref, wge3_ref, wgd1_ref, wgd2_ref, wgd3_ref,
               whe1_ref, whe2_ref, whe3_ref, whd1_ref, whd2_ref, whd3_ref,
               alpha_ref,
               h_ref, h1_ref, h2_ref, h3_ref, x1_ref, x2_ref, x3_ref):
    a1 = a1_ref[:]
    a2 = a2_ref[:]
    g = g_ref[:]
    h1 = _gcn3(x_ref[:], a1, wge1_ref[:], wge2_ref[:], wge3_ref[:])
    h2 = _gcn3(xm_ref[:], a2, wge1_ref[:], wge2_ref[:], wge3_ref[:])
    h3 = _gcn3(x_ref[:], g, whe1_ref[:], whe2_ref[:], whe3_ref[:])
    alpha = alpha_ref[0, 0]
    h = alpha * 0.5 * (h1 + h2) + (1.0 - alpha) * h3
    h_ref[:] = h
    h1_ref[:] = h1.astype(_BF)
    h2_ref[:] = h2.astype(_BF)
    h3_ref[:] = h3.astype(_BF)
    h_bf = h.astype(_BF)
    x1_ref[:] = _gcn3(h_bf, a1, wgd1_ref[:], wgd2_ref[:], wgd3_ref[:])
    x2_ref[:] = _gcn3(h_bf, a2, wgd1_ref[:], wgd2_ref[:], wgd3_ref[:])
    x3_ref[:] = _gcn3(h_bf, g, whd1_ref[:], whd2_ref[:], whd3_ref[:])


def _main(x_bf, xm_bf, a1_bf, a2_bf, g_bf, ws, alpha):
    f32 = jnp.float32
    out_shapes = (
        jax.ShapeDtypeStruct((N, 32), f32),
        jax.ShapeDtypeStruct((N, 32), _BF),
        jax.ShapeDtypeStruct((N, 32), _BF),
        jax.ShapeDtypeStruct((N, 32), _BF),
        jax.ShapeDtypeStruct((N, 256), f32),
        jax.ShapeDtypeStruct((N, 256), f32),
        jax.ShapeDtypeStruct((N, 256), f32),
    )
    from jax.experimental.pallas import tpu as pltpu
    return pl.pallas_call(
        _main_body,
        out_shape=out_shapes,
        compiler_params=pltpu.CompilerParams(
            vmem_limit_bytes=100 * 1024 * 1024),
    )(x_bf, xm_bf, a1_bf, a2_bf, g_bf, *ws, alpha.reshape(1, 1))


_S_BLK = 512


def _s_body(hi_ref, hj_ref, xi_ref, xj_ref, out_ref):
    t_enc = jnp.tanh(0.5 * _dot_nt(hi_ref[:], hj_ref[:]))
    t_dec = jnp.tanh(0.5 * _dot_nt(xi_ref[:].astype(_BF), xj_ref[:].astype(_BF)))
    out_ref[:] = 0.5 + 0.25 * (t_enc + t_dec)


def _s_avg(h_enc, x_dec):
    nb = N // _S_BLK
    kh = h_enc.shape[1]
    kx = x_dec.shape[1]
    return pl.pallas_call(
        _s_body,
        grid=(nb, nb),
        in_specs=[
            pl.BlockSpec((_S_BLK, kh), lambda i, j: (i, 0)),
            pl.BlockSpec((_S_BLK, kh), lambda i, j: (j, 0)),
            pl.BlockSpec((_S_BLK, kx), lambda i, j: (i, 0)),
            pl.BlockSpec((_S_BLK, kx), lambda i, j: (j, 0)),
        ],
        out_specs=pl.BlockSpec((_S_BLK, _S_BLK), lambda i, j: (i, j)),
        out_shape=jax.ShapeDtypeStruct((N, N), jnp.float32),
    )(h_enc, h_enc, x_dec, x_dec)


def kernel(x, x_mask, A1, A2, G, Wg_e1, Wg_e2, Wg_e3, Wg_d1, Wg_d2, Wg_d3,
           Wh_e1, Wh_e2, Wh_e3, Wh_d1, Wh_d2, Wh_d3, alpha):
    ws = (Wg_e1, Wg_e2, Wg_e3, Wg_d1, Wg_d2, Wg_d3,
          Wh_e1, Wh_e2, Wh_e3, Wh_d1, Wh_d2, Wh_d3)
    h, h1, h2, h3, x1, x2, x3 = _main(
        x.astype(_BF), x_mask.astype(_BF),
        A1.astype(_BF), A2.astype(_BF), G.astype(_BF), ws, alpha)
    s1 = _s_avg(h1, x1)
    s2 = _s_avg(h2, x2)
    s3 = _s_avg(h3, x3)
    return (h, s1, s2, s3, x1, x2, x3)
